# baseline (speedup 1.0000x reference)
import jax
import jax.numpy as jnp
from jax.experimental import pallas as pl
from jax.experimental.pallas import tpu as pltpu

IN_DIM = 2048
HID = 2048
N_EXPERTS = 64
TOKENS = 8192
BT = 512
QR = 512


def _gating_kernel(x_ref, w0_hbm, b0_ref, w1_hbm, b1_ref, w2_hbm, b2_ref,
                   w3_hbm, b3_ref, q_ref, idx_ref, raw_ref,
                   w0b, w1b, w2b, w3b, stg0, stg1, stg3, sems):
    first = pl.program_id(0) == 0

    nq = HID // QR
    pieces = []
    for src, dst in ((w0_hbm, w0b), (w1_hbm, w1b), (w2_hbm, w2b)):
        for qi in range(nq):
            pieces.append((src, dst, qi))
    cps = [
        pltpu.make_async_copy(src.at[pl.ds(qi * QR, QR), :],
                              stg0 if j % 2 == 0 else stg1,
                              sems.at[j])
        for j, (src, dst, qi) in enumerate(pieces)
    ]
    w3cp = pltpu.make_async_copy(w3_hbm, stg3, sems.at[len(pieces)])

    @pl.when(first)
    def _():
        cps[0].start()
        cps[1].start()
        w3cp.start()
        for j, (src, dst, qi) in enumerate(pieces):
            cps[j].wait()
            stg = stg0 if j % 2 == 0 else stg1
            dst[pl.ds(qi * QR, QR), :] = stg[...].astype(jnp.bfloat16)
            if j + 2 < len(pieces):
                cps[j + 2].start()
        w3cp.wait()
        w3b[...] = stg3[...].astype(jnp.bfloat16)

    def layer(h_in, w_ref, b_ref):
        acc = jnp.dot(h_in, w_ref[...], preferred_element_type=jnp.float32)
        return jnp.maximum(acc + b_ref[...], 0.0).astype(jnp.bfloat16)

    HB = BT // 2
    for p in range(2):
        rows = pl.ds(p * HB, HB)
        with jax.named_scope(f"xin{p}"):
            x = x_ref[rows, :].astype(jnp.bfloat16)
        with jax.named_scope(f"l0_{p}"):
            h = layer(x, w0b, b0_ref)
        with jax.named_scope(f"l1_{p}"):
            h = layer(h, w1b, b1_ref)
        with jax.named_scope(f"l2_{p}"):
            h = layer(h, w2b, b2_ref)
        with jax.named_scope(f"head{p}"):
            raw = jnp.dot(h, w3b[...], preferred_element_type=jnp.float32)
            raw = raw + b3_ref[...]
        with jax.named_scope(f"tail{p}"):
            m = jnp.max(raw, axis=1, keepdims=True)
            e = jnp.exp(raw - m)
            q = e / jnp.sum(e, axis=1, keepdims=True)
            q_ref[rows, :] = q
            raw_ref[0, :, rows] = raw.T
            mx = jnp.max(q, axis=1, keepdims=True)
            ii = jax.lax.broadcasted_iota(jnp.int32, q.shape, 1)
            idx_ref[rows] = jnp.min(jnp.where(q == mx, ii, N_EXPERTS), axis=1)


def kernel(points, W0, b0, W1, b1, W2, b2, W3, b3):
    nb = TOKENS // BT
    b0r = b0.reshape(1, HID)
    b1r = b1.reshape(1, HID)
    b2r = b2.reshape(1, HID)
    b3r = b3.reshape(1, N_EXPERTS)

    hbm = pl.BlockSpec(memory_space=pl.ANY)
    full = lambda shape: pl.BlockSpec(shape, lambda i: (0,) * len(shape))
    q, idx3, raw = pl.pallas_call(
        _gating_kernel,
        grid=(nb,),
        in_specs=[
            pl.BlockSpec((BT, IN_DIM), lambda i: (i, 0)),
            hbm, full((1, HID)),
            hbm, full((1, HID)),
            hbm, full((1, HID)),
            hbm, full((1, N_EXPERTS)),
        ],
        out_specs=[
            pl.BlockSpec((BT, N_EXPERTS), lambda i: (i, 0)),
            pl.BlockSpec((BT,), lambda i: (i,)),
            pl.BlockSpec((1, N_EXPERTS, BT), lambda i: (0, 0, i)),
        ],
        out_shape=[
            jax.ShapeDtypeStruct((TOKENS, N_EXPERTS), jnp.float32),
            jax.ShapeDtypeStruct((TOKENS,), jnp.int32),
            jax.ShapeDtypeStruct((1, N_EXPERTS, TOKENS), jnp.float32),
        ],
        scratch_shapes=[
            pltpu.MemorySpace.VMEM((IN_DIM, HID), jnp.bfloat16),
            pltpu.MemorySpace.VMEM((HID, HID), jnp.bfloat16),
            pltpu.MemorySpace.VMEM((HID, HID), jnp.bfloat16),
            pltpu.MemorySpace.VMEM((HID, N_EXPERTS), jnp.bfloat16),
            pltpu.MemorySpace.VMEM((QR, HID), jnp.float32),
            pltpu.MemorySpace.VMEM((QR, HID), jnp.float32),
            pltpu.MemorySpace.VMEM((HID, N_EXPERTS), jnp.float32),
            pltpu.SemaphoreType.DMA((32,)),
        ],
        compiler_params=pltpu.CompilerParams(
            dimension_semantics=("arbitrary",)),
    )(points, W0, b0r, W1, b1r, W2, b2r, W3, b3r)
    return (q, idx3, raw)


# speedup vs baseline: 1.0146x; 1.0146x over previous
import jax
import jax.numpy as jnp
from jax.experimental import pallas as pl
from jax.experimental.pallas import tpu as pltpu

IN_DIM = 2048
HID = 2048
N_EXPERTS = 64
TOKENS = 8192
BT = 512
QR = 512


def _gating_kernel(x_ref, w0_hbm, b0_ref, w1_hbm, b1_ref, w2_hbm, b2_ref,
                   w3_hbm, b3_ref, q_ref, idx_ref, raw_ref,
                   w0b, w1b, w2b, w3b, stg0, stg1, stg3, sems):
    first = pl.program_id(0) == 0

    nq = HID // QR
    pieces = []
    for src, dst in ((w0_hbm, w0b), (w1_hbm, w1b), (w2_hbm, w2b)):
        for qi in range(nq):
            pieces.append((src, dst, qi))
    cps = [
        pltpu.make_async_copy(src.at[pl.ds(qi * QR, QR), :],
                              stg0 if j % 2 == 0 else stg1,
                              sems.at[j])
        for j, (src, dst, qi) in enumerate(pieces)
    ]
    w3cp = pltpu.make_async_copy(w3_hbm, stg3, sems.at[len(pieces)])

    @pl.when(first)
    def _():
        cps[0].start()
        cps[1].start()
        w3cp.start()
        for j, (src, dst, qi) in enumerate(pieces):
            cps[j].wait()
            stg = stg0 if j % 2 == 0 else stg1
            dst[pl.ds(qi * QR, QR), :] = stg[...].astype(jnp.bfloat16)
            if j + 2 < len(pieces):
                cps[j + 2].start()
        w3cp.wait()
        w3b[...] = stg3[...].astype(jnp.bfloat16)

    def layer(h_in, w_ref, b_ref):
        acc = jnp.dot(h_in, w_ref[...], preferred_element_type=jnp.float32)
        return jnp.maximum(acc + b_ref[...], 0.0).astype(jnp.bfloat16)

    HB = BT // 2
    for p in range(2):
        rows = pl.ds(p * HB, HB)
        with jax.named_scope(f"xin{p}"):
            x = x_ref[rows, :].astype(jnp.bfloat16)
        with jax.named_scope(f"l0_{p}"):
            h = layer(x, w0b, b0_ref)
        with jax.named_scope(f"l1_{p}"):
            h = layer(h, w1b, b1_ref)
        with jax.named_scope(f"l2_{p}"):
            h = layer(h, w2b, b2_ref)
        with jax.named_scope(f"head{p}"):
            raw = jnp.dot(h, w3b[...], preferred_element_type=jnp.float32)
            raw = raw + b3_ref[...]
        with jax.named_scope(f"tail{p}"):
            m = jnp.max(raw, axis=1, keepdims=True)
            e = jnp.exp(raw - m)
            q = e / jnp.sum(e, axis=1, keepdims=True)
            q_ref[rows, :] = q
            raw_ref[0, :, rows] = raw.T
            ii = jax.lax.broadcasted_iota(jnp.int32, raw.shape, 1)
            idx_ref[rows] = jnp.min(jnp.where(raw == m, ii, N_EXPERTS), axis=1)


def kernel(points, W0, b0, W1, b1, W2, b2, W3, b3):
    nb = TOKENS // BT
    b0r = b0.reshape(1, HID)
    b1r = b1.reshape(1, HID)
    b2r = b2.reshape(1, HID)
    b3r = b3.reshape(1, N_EXPERTS)

    hbm = pl.BlockSpec(memory_space=pl.ANY)
    full = lambda shape: pl.BlockSpec(shape, lambda i: (0,) * len(shape))
    q, idx3, raw = pl.pallas_call(
        _gating_kernel,
        grid=(nb,),
        in_specs=[
            pl.BlockSpec((BT, IN_DIM), lambda i: (i, 0)),
            hbm, full((1, HID)),
            hbm, full((1, HID)),
            hbm, full((1, HID)),
            hbm, full((1, N_EXPERTS)),
        ],
        out_specs=[
            pl.BlockSpec((BT, N_EXPERTS), lambda i: (i, 0)),
            pl.BlockSpec((BT,), lambda i: (i,)),
            pl.BlockSpec((1, N_EXPERTS, BT), lambda i: (0, 0, i)),
        ],
        out_shape=[
            jax.ShapeDtypeStruct((TOKENS, N_EXPERTS), jnp.float32),
            jax.ShapeDtypeStruct((TOKENS,), jnp.int32),
            jax.ShapeDtypeStruct((1, N_EXPERTS, TOKENS), jnp.float32),
        ],
        scratch_shapes=[
            pltpu.MemorySpace.VMEM((IN_DIM, HID), jnp.bfloat16),
            pltpu.MemorySpace.VMEM((HID, HID), jnp.bfloat16),
            pltpu.MemorySpace.VMEM((HID, HID), jnp.bfloat16),
            pltpu.MemorySpace.VMEM((HID, N_EXPERTS), jnp.bfloat16),
            pltpu.MemorySpace.VMEM((QR, HID), jnp.float32),
            pltpu.MemorySpace.VMEM((QR, HID), jnp.float32),
            pltpu.MemorySpace.VMEM((HID, N_EXPERTS), jnp.float32),
            pltpu.SemaphoreType.DMA((32,)),
        ],
        compiler_params=pltpu.CompilerParams(
            dimension_semantics=("arbitrary",)),
    )(points, W0, b0r, W1, b1r, W2, b2r, W3, b3r)
    return (q, idx3, raw)


# speedup vs baseline: 1.0147x; 1.0001x over previous
import jax
import jax.numpy as jnp
from jax.experimental import pallas as pl
from jax.experimental.pallas import tpu as pltpu

IN_DIM = 2048
HID = 2048
N_EXPERTS = 64
TOKENS = 8192
BT = 512
QR = 512


def _gating_kernel(x_ref, w0_hbm, b0_ref, w1_hbm, b1_ref, w2_hbm, b2_ref,
                   w3_hbm, b3_ref, q_ref, idx_ref, raw_ref,
                   w0b, w1b, w2b, w3b, stg0, stg1, stg3, sems):
    first = pl.program_id(0) == 0

    nq = HID // QR
    pieces = []
    for src, dst in ((w0_hbm, w0b), (w1_hbm, w1b), (w2_hbm, w2b)):
        for qi in range(nq):
            pieces.append((src, dst, qi))
    cps = [
        pltpu.make_async_copy(src.at[pl.ds(qi * QR, QR), :],
                              stg0 if j % 2 == 0 else stg1,
                              sems.at[j])
        for j, (src, dst, qi) in enumerate(pieces)
    ]
    w3cp = pltpu.make_async_copy(w3_hbm, stg3, sems.at[len(pieces)])

    @pl.when(first)
    def _():
        cps[0].start()
        cps[1].start()
        w3cp.start()
        for j, (src, dst, qi) in enumerate(pieces):
            cps[j].wait()
            stg = stg0 if j % 2 == 0 else stg1
            dst[pl.ds(qi * QR, QR), :] = stg[...].astype(jnp.bfloat16)
            if j + 2 < len(pieces):
                cps[j + 2].start()
        w3cp.wait()
        w3b[...] = stg3[...].astype(jnp.bfloat16)

    def layer(h_in, w_ref, b_ref):
        acc = jnp.dot(h_in, w_ref[...], preferred_element_type=jnp.float32)
        return jnp.maximum(acc + b_ref[...], 0.0).astype(jnp.bfloat16)

    HB = BT // 2
    for p in range(2):
        rows = pl.ds(p * HB, HB)
        x = x_ref[rows, :].astype(jnp.bfloat16)
        h = layer(x, w0b, b0_ref)
        h = layer(h, w1b, b1_ref)
        h = layer(h, w2b, b2_ref)
        raw = jnp.dot(h, w3b[...], preferred_element_type=jnp.float32)
        raw = raw + b3_ref[...]
        m = jnp.max(raw, axis=1, keepdims=True)
        e = jnp.exp(raw - m)
        q = e / jnp.sum(e, axis=1, keepdims=True)
        q_ref[rows, :] = q
        raw_ref[0, :, rows] = raw.T
        ii = jax.lax.broadcasted_iota(jnp.int32, raw.shape, 1)
        idx_ref[rows] = jnp.min(jnp.where(raw == m, ii, N_EXPERTS), axis=1)


def kernel(points, W0, b0, W1, b1, W2, b2, W3, b3):
    nb = TOKENS // BT
    b0r = b0.reshape(1, HID)
    b1r = b1.reshape(1, HID)
    b2r = b2.reshape(1, HID)
    b3r = b3.reshape(1, N_EXPERTS)

    hbm = pl.BlockSpec(memory_space=pl.ANY)
    full = lambda shape: pl.BlockSpec(shape, lambda i: (0,) * len(shape))
    q, idx3, raw = pl.pallas_call(
        _gating_kernel,
        grid=(nb,),
        in_specs=[
            pl.BlockSpec((BT, IN_DIM), lambda i: (i, 0)),
            hbm, full((1, HID)),
            hbm, full((1, HID)),
            hbm, full((1, HID)),
            hbm, full((1, N_EXPERTS)),
        ],
        out_specs=[
            pl.BlockSpec((BT, N_EXPERTS), lambda i: (i, 0)),
            pl.BlockSpec((BT,), lambda i: (i,)),
            pl.BlockSpec((1, N_EXPERTS, BT), lambda i: (0, 0, i)),
        ],
        out_shape=[
            jax.ShapeDtypeStruct((TOKENS, N_EXPERTS), jnp.float32),
            jax.ShapeDtypeStruct((TOKENS,), jnp.int32),
            jax.ShapeDtypeStruct((1, N_EXPERTS, TOKENS), jnp.float32),
        ],
        scratch_shapes=[
            pltpu.MemorySpace.VMEM((IN_DIM, HID), jnp.bfloat16),
            pltpu.MemorySpace.VMEM((HID, HID), jnp.bfloat16),
            pltpu.MemorySpace.VMEM((HID, HID), jnp.bfloat16),
            pltpu.MemorySpace.VMEM((HID, N_EXPERTS), jnp.bfloat16),
            pltpu.MemorySpace.VMEM((QR, HID), jnp.float32),
            pltpu.MemorySpace.VMEM((QR, HID), jnp.float32),
            pltpu.MemorySpace.VMEM((HID, N_EXPERTS), jnp.float32),
            pltpu.SemaphoreType.DMA((32,)),
        ],
        compiler_params=pltpu.CompilerParams(
            dimension_semantics=("arbitrary",)),
    )(points, W0, b0r, W1, b1r, W2, b2r, W3, b3r)
    return (q, idx3, raw)
